# baseline (speedup 1.0000x reference)
import jax
import jax.numpy as jnp
from jax import lax
from jax.experimental import pallas as pl
from jax.experimental.pallas import tpu as pltpu

DIM = 128
NUM_MODELS = 64
TEXT_DIM = 1536
BATCH = 4096
ROWS = BATCH // 128
NCHUNK = 4
RCH = DIM // NCHUNK


def _tc_body(ids_ref, prompt_ref, p_ref, wt_hbm, wcls_ref, out_ref,
             wt_buf, sems):
    def chunk_copy(k, slot):
        return pltpu.make_async_copy(
            wt_hbm.at[pl.ds(k * RCH, RCH), :], wt_buf.at[slot], sems.at[slot])

    chunk_copy(0, 0).start()
    pe_parts = []
    for k in range(NCHUNK):
        slot = k % 2
        if k + 1 < NCHUNK:
            chunk_copy(k + 1, (k + 1) % 2).start()
        chunk_copy(k, slot).wait()
        pe_parts.append(lax.dot_general(
            prompt_ref[...], wt_buf[slot],
            dimension_numbers=(((1,), (1,)), ((), ())),
            preferred_element_type=jnp.float32,
        ))
    pe = jnp.concatenate(pe_parts, axis=1)
    w = pe * wcls_ref[...]
    p = p_ref[...]
    srow = lax.dot_general(
        w, p, dimension_numbers=(((1,), (1,)), ((), ())),
        preferred_element_type=jnp.float32,
    )
    n2row = lax.dot_general(
        jnp.ones((1, DIM), jnp.float32), p * p,
        dimension_numbers=(((1,), (1,)), ((), ())),
        preferred_element_type=jnp.float32,
    )
    s = srow / jnp.maximum(jnp.sqrt(n2row), 1e-12)
    sb = jnp.broadcast_to(s, (ROWS, NUM_MODELS))
    out_ref[...] = jnp.take_along_axis(sb, ids_ref[...], axis=1)


def kernel(model_id, prompt_embed, P, W_text, W_cls):
    out = pl.pallas_call(
        _tc_body,
        out_shape=jax.ShapeDtypeStruct((ROWS, 128), jnp.float32),
        in_specs=[
            pl.BlockSpec(memory_space=pltpu.VMEM),
            pl.BlockSpec(memory_space=pltpu.VMEM),
            pl.BlockSpec(memory_space=pltpu.VMEM),
            pl.BlockSpec(memory_space=pltpu.MemorySpace.HBM),
            pl.BlockSpec(memory_space=pltpu.VMEM),
        ],
        scratch_shapes=[
            pltpu.VMEM((2, RCH, TEXT_DIM), jnp.float32),
            pltpu.SemaphoreType.DMA((2,)),
        ],
    )(model_id.astype(jnp.int32).reshape(ROWS, 128),
      prompt_embed.reshape(1, TEXT_DIM), P, W_text, W_cls)
    return out.reshape(BATCH)


# speedup vs baseline: 1.4799x; 1.4799x over previous
import jax
import jax.numpy as jnp
from jax import lax
from jax.experimental import pallas as pl

DIM = 128
NUM_MODELS = 64
TEXT_DIM = 1536
BATCH = 4096
ROWS = BATCH // 128


def _tc_body(ids_ref, prompt_ref, p_ref, wt_ref, wcls_ref, out_ref):
    pe = lax.dot_general(
        prompt_ref[...], wt_ref[...],
        dimension_numbers=(((1,), (1,)), ((), ())),
        preferred_element_type=jnp.float32,
    )
    w = pe * wcls_ref[...]
    p = p_ref[...]
    srow = lax.dot_general(
        w, p, dimension_numbers=(((1,), (1,)), ((), ())),
        preferred_element_type=jnp.float32,
    )
    n2row = lax.dot_general(
        jnp.ones((1, DIM), jnp.float32), p * p,
        dimension_numbers=(((1,), (1,)), ((), ())),
        preferred_element_type=jnp.float32,
    )
    s = srow / jnp.maximum(jnp.sqrt(n2row), 1e-12)
    sb = jnp.broadcast_to(s, (ROWS, NUM_MODELS))
    out_ref[...] = jnp.take_along_axis(sb, ids_ref[...], axis=1)


def kernel(model_id, prompt_embed, P, W_text, W_cls):
    out = pl.pallas_call(
        _tc_body,
        out_shape=jax.ShapeDtypeStruct((ROWS, 128), jnp.float32),
    )(model_id.astype(jnp.int32).reshape(ROWS, 128),
      prompt_embed.reshape(1, TEXT_DIM), P, W_text, W_cls)
    return out.reshape(BATCH)
